# baseline (speedup 1.0000x reference)
import functools

import jax
import jax.numpy as jnp
from jax import lax
from jax.experimental import pallas as pl
from jax.experimental.pallas import tpu as pltpu
from jax.experimental.pallas import tpu_sc as plsc

_E = 64
_K = 8
_NC = 2
_NS = 16
_L = 16


def _mm_body(x_ref, w_ref, lt_ref):
    x = x_ref[...]
    w = w_ref[...]
    lt_ref[...] = lax.dot_general(
        w, x, (((1,), (1,)), ((), ())), preferred_element_type=jnp.float32
    )


def _tc_logits(x, gate_W):
    tokens, dim = x.shape
    bt = 4096
    return pl.pallas_call(
        _mm_body,
        grid=(tokens // bt,),
        in_specs=[
            pl.BlockSpec((bt, dim), lambda i: (i, 0)),
            pl.BlockSpec((_E, dim), lambda i: (0, 0)),
        ],
        out_specs=pl.BlockSpec((_E, bt), lambda i: (0, i)),
        out_shape=jax.ShapeDtypeStruct((_E, tokens), jnp.float32),
        compiler_params=pltpu.CompilerParams(
            dimension_semantics=("parallel",),
        ),
    )(x, gate_W)


def _sc_topk_call(lt):
    tokens = lt.shape[1]
    tpw = tokens // (_NC * _NS)
    ngrp = tpw // _L
    mesh = plsc.VectorSubcoreMesh(core_axis_name="c", subcore_axis_name="s")

    @functools.partial(
        pl.kernel,
        mesh=mesh,
        out_type=[
            jax.ShapeDtypeStruct((_K, tokens), jnp.float32),
            jax.ShapeDtypeStruct((_K, tokens), jnp.int32),
        ],
        scratch_types=[
            pltpu.VMEM((_E, tpw), jnp.float32),
            pltpu.VMEM((_K, tpw), jnp.float32),
            pltpu.VMEM((_K, tpw), jnp.int32),
        ],
    )
    def _sc_topk(lt_hbm, wts_hbm, idx_hbm, lt_v, w_v, i_v):
        wid = lax.axis_index("s") * _NC + lax.axis_index("c")
        base = wid * tpw
        pltpu.sync_copy(lt_hbm.at[:, pl.ds(base, tpw)], lt_v)

        def group_body(g, _):
            gb = g * _L
            neg = jnp.full((_L,), -jnp.inf, jnp.float32)

            def round_body(j, carry):
                pv, pi = carry

                def expert_body(e, acc):
                    bm, bi = acc
                    v = lt_v[e, pl.ds(gb, _L)]
                    ef = jnp.full((_L,), 1.0, jnp.float32) * e.astype(jnp.float32)
                    elig = (v < pv) | ((v == pv) & (ef > pi))
                    cand = jnp.where(elig, v, neg)
                    upd = cand > bm
                    return (jnp.where(upd, cand, bm), jnp.where(upd, ef, bi))

                bm0 = neg
                bi0 = jnp.full((_L,), -1.0, jnp.float32)
                bm, bi = lax.fori_loop(0, _E, expert_body, (bm0, bi0))
                w_v[j, pl.ds(gb, _L)] = bm
                i_v[j, pl.ds(gb, _L)] = bi.astype(jnp.int32)
                return (bm, bi)

            inf0 = jnp.full((_L,), jnp.inf, jnp.float32)
            lax.fori_loop(0, _K, round_body, (inf0, jnp.full((_L,), -1.0, jnp.float32)))

            m = w_v[0, pl.ds(gb, _L)]
            es = []
            tot = jnp.full((_L,), 0.0, jnp.float32)
            for j in range(_K):
                ej = jnp.exp(w_v[j, pl.ds(gb, _L)] - m)
                es.append(ej)
                tot = tot + ej
            for j in range(_K):
                w_v[j, pl.ds(gb, _L)] = es[j] / tot
            return 0

        lax.fori_loop(0, ngrp, group_body, 0)
        pltpu.sync_copy(w_v, wts_hbm.at[:, pl.ds(base, tpw)])
        pltpu.sync_copy(i_v, idx_hbm.at[:, pl.ds(base, tpw)])

    return _sc_topk(lt)


def kernel(x, gate_W):
    lt = _tc_logits(x, gate_W)
    wts_t, idx_t = _sc_topk_call(lt)
    return wts_t.T, idx_t.T


# speedup vs baseline: 4.9328x; 4.9328x over previous
import jax
import jax.numpy as jnp
from jax import lax
from jax.experimental import pallas as pl
from jax.experimental.pallas import tpu as pltpu

_E = 64
_K = 8
_C = 128


def _topk_chunk(blk, lane_e):
    work = blk
    vals = []
    idxs = []
    for j in range(_K):
        m = jnp.max(work, axis=0, keepdims=True)
        key = jnp.where(work == m, lane_e, float(_E))
        ixf = jnp.min(key, axis=0, keepdims=True)
        vals.append(m)
        idxs.append(ixf)
        if j < _K - 1:
            work = jnp.where(lane_e == ixf, -jnp.inf, work)
    v = jnp.concatenate(vals, axis=0)
    ixf = jnp.concatenate(idxs, axis=0)
    e = jnp.exp(v - v[0:1, :])
    wts = e / jnp.sum(e, axis=0, keepdims=True)
    return wts, ixf.astype(jnp.int32)


def _gate_body(x_ref, w_ref, wts_ref, idx_ref):
    x = x_ref[...]
    w = w_ref[...]
    lt = lax.dot_general(
        w, x, (((1,), (1,)), ((), ())), preferred_element_type=jnp.float32
    )
    lane_e = lax.broadcasted_iota(jnp.int32, (_E, _C), 0).astype(jnp.float32)
    bt = lt.shape[1]
    for c in range(bt // _C):
        blk = lt[:, c * _C:(c + 1) * _C]
        wts_c, idx_c = _topk_chunk(blk, lane_e)
        wts_ref[:, pl.ds(c * _C, _C)] = wts_c
        idx_ref[:, pl.ds(c * _C, _C)] = idx_c


def kernel(x, gate_W):
    tokens, dim = x.shape
    bt = 4096
    grid = (tokens // bt,)
    wts_t, idx_t = pl.pallas_call(
        _gate_body,
        grid=grid,
        in_specs=[
            pl.BlockSpec((bt, dim), lambda i: (i, 0)),
            pl.BlockSpec((_E, dim), lambda i: (0, 0)),
        ],
        out_specs=[
            pl.BlockSpec((_K, bt), lambda i: (0, i)),
            pl.BlockSpec((_K, bt), lambda i: (0, i)),
        ],
        out_shape=[
            jax.ShapeDtypeStruct((_K, tokens), jnp.float32),
            jax.ShapeDtypeStruct((_K, tokens), jnp.int32),
        ],
        compiler_params=pltpu.CompilerParams(
            dimension_semantics=("parallel",),
        ),
    )(x, gate_W)
    return wts_t.T, idx_t.T
